# baseline (speedup 1.0000x reference)
import functools

import jax
import jax.numpy as jnp
from jax import lax
from jax.experimental import pallas as pl
from jax.experimental.pallas import tpu as pltpu
from jax.experimental.pallas import tpu_sc as plsc

B, F, N, K = 8, 64, 256, 512
NC, NS, L = 2, 16, 16
NW = NC * NS
TOK = B * N
TPW = TOK // NW


def _split(a):
    hi = a.astype(jnp.bfloat16)
    lo = (a - hi.astype(jnp.float32)).astype(jnp.bfloat16)
    return hi, lo


def _bdot(a, b, dims):
    return lax.dot_general(a, b, (dims, ((), ())),
                           preferred_element_type=jnp.float32)


def _dist_body(x_ref, emb_ref, dist_ref, idx_ref):
    xb = x_ref[0]
    emb = emb_ref[...]
    xh, xl = _split(xb)
    eh, el = _split(emb)
    cd = ((1,), (0,))
    dotT = (_bdot(eh, xh, cd) + _bdot(eh, xl, cd)
            + _bdot(el, xh, cd))
    c2 = jnp.sum(emb * emb, axis=1, keepdims=True)
    x2 = jnp.sum(xb * xb, axis=0, keepdims=True)
    gT = c2 - 2.0 * dotT
    dist_ref[0] = (gT + x2).T
    minv = jnp.min(gT, axis=0, keepdims=True)
    iota = lax.broadcasted_iota(jnp.int32, (K, N), 0)
    idx_ref[0] = jnp.min(jnp.where(gT == minv, iota, K), axis=0, keepdims=True)


_sc_mesh = plsc.VectorSubcoreMesh(core_axis_name="c", subcore_axis_name="s")


@functools.partial(
    pl.kernel,
    mesh=_sc_mesh,
    out_type=jax.ShapeDtypeStruct((B * F, N), jnp.float32),
    scratch_types=[
        pltpu.VMEM((TPW,), jnp.int32),
        pltpu.VMEM((TPW, F), jnp.float32),
        pltpu.VMEM((F, TPW), jnp.float32),
        pltpu.SemaphoreType.DMA,
    ],
    compiler_params=pltpu.CompilerParams(use_tc_tiling_on_sc=False,
                                         needs_layout_passes=False),
)
def _sc_gather_t(table_hbm, idx_hbm, out_hbm, idx_v, rows_v, trans_v, sem):
    wid = lax.axis_index("s") * NC + lax.axis_index("c")
    base = wid * TPW
    b = wid // (N // TPW)
    c0 = (wid % (N // TPW)) * TPW
    pltpu.sync_copy(idx_hbm.at[pl.ds(base, TPW)], idx_v)
    pltpu.async_copy(table_hbm.at[idx_v], rows_v, sem).wait()
    lane = lax.iota(jnp.int32, L)
    rows = [lane + (L * j) for j in range(TPW // L)]
    for f in range(F):
        col = jnp.full((L,), f, jnp.int32)
        for j in range(TPW // L):
            trans_v[f, pl.ds(L * j, L)] = plsc.load_gather(
                rows_v, [rows[j], col])
    pltpu.sync_copy(trans_v, out_hbm.at[pl.ds(b * F, F), pl.ds(c0, TPW)])


def kernel(x, emb_weight):
    dist, idx = pl.pallas_call(
        _dist_body,
        grid=(B,),
        in_specs=[
            pl.BlockSpec((1, F, N), lambda b: (b, 0, 0)),
            pl.BlockSpec((K, F), lambda b: (0, 0)),
        ],
        out_specs=[
            pl.BlockSpec((1, N, K), lambda b: (b, 0, 0)),
            pl.BlockSpec((1, 1, N), lambda b: (b, 0, 0)),
        ],
        out_shape=[
            jax.ShapeDtypeStruct((B, N, K), jnp.float32),
            jax.ShapeDtypeStruct((B, 1, N), jnp.int32),
        ],
    )(x, emb_weight)
    q = _sc_gather_t(emb_weight, idx.reshape(TOK))
    return q.reshape(B, F, N), dist


# speedup vs baseline: 1.0005x; 1.0005x over previous
import functools

import jax
import jax.numpy as jnp
from jax import lax
from jax.experimental import pallas as pl
from jax.experimental.pallas import tpu as pltpu
from jax.experimental.pallas import tpu_sc as plsc

B, F, N, K = 8, 64, 256, 512
NC, NS, L = 2, 16, 16
NW = NC * NS
TOK = B * N
TPW = TOK // NW


def _split(a):
    hi = a.astype(jnp.bfloat16)
    lo = (a - hi.astype(jnp.float32)).astype(jnp.bfloat16)
    return hi, lo


def _bdot(a, b, dims):
    return lax.dot_general(a, b, (dims, ((), ())),
                           preferred_element_type=jnp.float32)


def _dist_body(x_ref, emb_ref, dist_ref, idx_ref):
    xb = x_ref[0]
    emb = emb_ref[...]
    xh, xl = _split(xb)
    eh, el = _split(emb)
    cd = ((1,), (0,))
    dotT = (_bdot(eh, xh, cd) + _bdot(eh, xl, cd)
            + _bdot(el, xh, cd))
    c2 = jnp.sum(emb * emb, axis=1, keepdims=True)
    x2 = jnp.sum(xb * xb, axis=0, keepdims=True)
    gT = c2 - 2.0 * dotT
    dist_ref[0] = (gT + x2).T
    minv = jnp.min(gT, axis=0, keepdims=True)
    iota = lax.broadcasted_iota(jnp.int32, (K, N), 0)
    idx_ref[0] = jnp.min(jnp.where(gT == minv, iota, K), axis=0, keepdims=True)


_sc_mesh = plsc.VectorSubcoreMesh(core_axis_name="c", subcore_axis_name="s")


@functools.partial(
    pl.kernel,
    mesh=_sc_mesh,
    out_type=jax.ShapeDtypeStruct((B * F, N), jnp.float32),
    scratch_types=[
        pltpu.VMEM((TPW,), jnp.int32),
        pltpu.VMEM((TPW, F), jnp.float32),
        pltpu.VMEM((F, TPW), jnp.float32),
        pltpu.SemaphoreType.DMA,
    ],
    compiler_params=pltpu.CompilerParams(use_tc_tiling_on_sc=False,
                                         needs_layout_passes=False,
                                         skip_device_barrier=True),
)
def _sc_gather_t(table_hbm, idx_hbm, out_hbm, idx_v, rows_v, trans_v, sem):
    wid = lax.axis_index("s") * NC + lax.axis_index("c")
    base = wid * TPW
    b = wid // (N // TPW)
    c0 = (wid % (N // TPW)) * TPW
    pltpu.sync_copy(idx_hbm.at[pl.ds(base, TPW)], idx_v)
    pltpu.async_copy(table_hbm.at[idx_v], rows_v, sem).wait()
    lane = lax.iota(jnp.int32, L)
    rows = [lane + (L * j) for j in range(TPW // L)]
    for f in range(F):
        col = jnp.full((L,), f, jnp.int32)
        for j in range(TPW // L):
            trans_v[f, pl.ds(L * j, L)] = plsc.load_gather(
                rows_v, [rows[j], col])
    pltpu.sync_copy(trans_v, out_hbm.at[pl.ds(b * F, F), pl.ds(c0, TPW)])


def kernel(x, emb_weight):
    dist, idx = pl.pallas_call(
        _dist_body,
        grid=(B,),
        in_specs=[
            pl.BlockSpec((1, F, N), lambda b: (b, 0, 0)),
            pl.BlockSpec((K, F), lambda b: (0, 0)),
        ],
        out_specs=[
            pl.BlockSpec((1, N, K), lambda b: (b, 0, 0)),
            pl.BlockSpec((1, 1, N), lambda b: (b, 0, 0)),
        ],
        out_shape=[
            jax.ShapeDtypeStruct((B, N, K), jnp.float32),
            jax.ShapeDtypeStruct((B, 1, N), jnp.int32),
        ],
    )(x, emb_weight)
    q = _sc_gather_t(emb_weight, idx.reshape(TOK))
    return q.reshape(B, F, N), dist


# speedup vs baseline: 1.0028x; 1.0023x over previous
import functools

import jax
import jax.numpy as jnp
from jax import lax
from jax.experimental import pallas as pl
from jax.experimental.pallas import tpu as pltpu
from jax.experimental.pallas import tpu_sc as plsc

B, F, N, K = 8, 64, 256, 512
NC, NS, L = 2, 16, 16
NW = NC * NS
TOK = B * N
TPW = TOK // NW


def _split(a):
    hi = a.astype(jnp.bfloat16)
    lo = (a - hi.astype(jnp.float32)).astype(jnp.bfloat16)
    return hi, lo


def _bdot(a, b, dims):
    return lax.dot_general(a, b, (dims, ((), ())),
                           preferred_element_type=jnp.float32)


def _gt(x_ref, emb_ref):
    xb = x_ref[0]
    emb = emb_ref[...]
    xh, xl = _split(xb)
    eh, el = _split(emb)
    cd = ((1,), (0,))
    dotT = (_bdot(eh, xh, cd) + _bdot(eh, xl, cd)
            + _bdot(el, xh, cd))
    c2 = jnp.sum(emb * emb, axis=1, keepdims=True)
    return c2 - 2.0 * dotT


def _argmin_body(x_ref, emb_ref, idx_ref):
    gT = _gt(x_ref, emb_ref)
    minv = jnp.min(gT, axis=0, keepdims=True)
    iota = lax.broadcasted_iota(jnp.int32, (K, N), 0)
    idx_ref[0] = jnp.min(jnp.where(gT == minv, iota, K), axis=0, keepdims=True)


def _dist_body(x_ref, emb_ref, dist_ref):
    xb = x_ref[0]
    x2 = jnp.sum(xb * xb, axis=0, keepdims=True)
    dist_ref[0] = (_gt(x_ref, emb_ref) + x2).T


_sc_mesh = plsc.VectorSubcoreMesh(core_axis_name="c", subcore_axis_name="s")


@functools.partial(
    pl.kernel,
    mesh=_sc_mesh,
    out_type=jax.ShapeDtypeStruct((B * F, N), jnp.float32),
    scratch_types=[
        pltpu.VMEM((TPW,), jnp.int32),
        pltpu.VMEM((TPW, F), jnp.float32),
        pltpu.VMEM((F, TPW), jnp.float32),
        pltpu.SemaphoreType.DMA,
    ],
    compiler_params=pltpu.CompilerParams(use_tc_tiling_on_sc=False,
                                         needs_layout_passes=False),
)
def _sc_gather_t(table_hbm, idx_hbm, out_hbm, idx_v, rows_v, trans_v, sem):
    wid = lax.axis_index("s") * NC + lax.axis_index("c")
    base = wid * TPW
    b = wid // (N // TPW)
    c0 = (wid % (N // TPW)) * TPW
    pltpu.sync_copy(idx_hbm.at[pl.ds(base, TPW)], idx_v)
    pltpu.async_copy(table_hbm.at[idx_v], rows_v, sem).wait()
    lane = lax.iota(jnp.int32, L)
    rows = [lane + (L * j) for j in range(TPW // L)]
    for f in range(F):
        col = jnp.full((L,), f, jnp.int32)
        for j in range(TPW // L):
            trans_v[f, pl.ds(L * j, L)] = plsc.load_gather(
                rows_v, [rows[j], col])
    pltpu.sync_copy(trans_v, out_hbm.at[pl.ds(b * F, F), pl.ds(c0, TPW)])


def kernel(x, emb_weight):
    idx = pl.pallas_call(
        _argmin_body,
        grid=(B,),
        in_specs=[
            pl.BlockSpec((1, F, N), lambda b: (b, 0, 0)),
            pl.BlockSpec((K, F), lambda b: (0, 0)),
        ],
        out_specs=pl.BlockSpec((1, 1, N), lambda b: (b, 0, 0)),
        out_shape=jax.ShapeDtypeStruct((B, 1, N), jnp.int32),
    )(x, emb_weight)
    q = _sc_gather_t(emb_weight, idx.reshape(TOK))
    dist = pl.pallas_call(
        _dist_body,
        grid=(B,),
        in_specs=[
            pl.BlockSpec((1, F, N), lambda b: (b, 0, 0)),
            pl.BlockSpec((K, F), lambda b: (0, 0)),
        ],
        out_specs=pl.BlockSpec((1, N, K), lambda b: (b, 0, 0)),
        out_shape=jax.ShapeDtypeStruct((B, N, K), jnp.float32),
    )(x, emb_weight)
    return q.reshape(B, F, N), dist


# speedup vs baseline: 1.0056x; 1.0028x over previous
import functools

import jax
import jax.numpy as jnp
from jax import lax
from jax.experimental import pallas as pl
from jax.experimental.pallas import tpu as pltpu
from jax.experimental.pallas import tpu_sc as plsc

B, F, N, K = 8, 64, 256, 512
NC, NS, L = 2, 16, 16
NW = NC * NS
TOK = B * N
TPW = TOK // NW


def _split(a):
    hi = a.astype(jnp.bfloat16)
    lo = (a - hi.astype(jnp.float32)).astype(jnp.bfloat16)
    return hi, lo


def _bdot(a, b, dims):
    return lax.dot_general(a, b, (dims, ((), ())),
                           preferred_element_type=jnp.float32)


def _gt(x_ref, emb_ref):
    xb = x_ref[0]
    emb = emb_ref[...]
    xh, xl = _split(xb)
    eh, el = _split(emb)
    cd = ((1,), (0,))
    dotT = (_bdot(eh, xh, cd) + _bdot(eh, xl, cd)
            + _bdot(el, xh, cd))
    c2 = jnp.sum(emb * emb, axis=1, keepdims=True)
    return c2 - 2.0 * dotT


def _argmin_body(x_ref, emb_ref, idx_ref):
    gT = _gt(x_ref, emb_ref)
    minv = jnp.min(gT, axis=0, keepdims=True)
    iota = lax.broadcasted_iota(jnp.int32, (K, N), 0)
    idx_ref[0] = jnp.min(jnp.where(gT == minv, iota, K), axis=0, keepdims=True)


def _dist_body(x_ref, emb_ref, dist_ref):
    xb = x_ref[0]
    x2 = jnp.sum(xb * xb, axis=0, keepdims=True)
    dist_ref[0] = (_gt(x_ref, emb_ref) + x2).T


_sc_mesh = plsc.VectorSubcoreMesh(core_axis_name="c", subcore_axis_name="s")


@functools.partial(
    pl.kernel,
    mesh=_sc_mesh,
    out_type=jax.ShapeDtypeStruct((B * F, N), jnp.float32),
    scratch_types=[
        pltpu.VMEM((TPW,), jnp.int32),
        pltpu.VMEM((TPW, F), jnp.float32),
        pltpu.VMEM((F, TPW), jnp.float32),
        pltpu.SemaphoreType.DMA,
    ],
    compiler_params=pltpu.CompilerParams(use_tc_tiling_on_sc=False,
                                         needs_layout_passes=False),
)
def _sc_gather_t(table_hbm, idx_hbm, out_hbm, idx_v, rows_v, trans_v, sem):
    wid = lax.axis_index("s") * NC + lax.axis_index("c")
    base = wid * TPW
    b = wid // (N // TPW)
    c0 = (wid % (N // TPW)) * TPW
    pltpu.sync_copy(idx_hbm.at[pl.ds(base, TPW)], idx_v)
    pltpu.async_copy(table_hbm.at[idx_v], rows_v, sem).wait()
    lane = lax.iota(jnp.int32, L)
    rows = [lane + (L * j) for j in range(TPW // L)]

    def _tr(f, _):
        col = jnp.zeros((L,), jnp.int32) + f
        for j in range(TPW // L):
            trans_v[f, pl.ds(L * j, L)] = plsc.load_gather(
                rows_v, [rows[j], col])
        return _

    lax.fori_loop(0, F, _tr, None)
    pltpu.sync_copy(trans_v, out_hbm.at[pl.ds(b * F, F), pl.ds(c0, TPW)])


def kernel(x, emb_weight):
    idx = pl.pallas_call(
        _argmin_body,
        grid=(B,),
        in_specs=[
            pl.BlockSpec((1, F, N), lambda b: (b, 0, 0)),
            pl.BlockSpec((K, F), lambda b: (0, 0)),
        ],
        out_specs=pl.BlockSpec((1, 1, N), lambda b: (b, 0, 0)),
        out_shape=jax.ShapeDtypeStruct((B, 1, N), jnp.int32),
    )(x, emb_weight)
    q = _sc_gather_t(emb_weight, idx.reshape(TOK))
    dist = pl.pallas_call(
        _dist_body,
        grid=(B,),
        in_specs=[
            pl.BlockSpec((1, F, N), lambda b: (b, 0, 0)),
            pl.BlockSpec((K, F), lambda b: (0, 0)),
        ],
        out_specs=pl.BlockSpec((1, N, K), lambda b: (b, 0, 0)),
        out_shape=jax.ShapeDtypeStruct((B, N, K), jnp.float32),
    )(x, emb_weight)
    return q.reshape(B, F, N), dist


# speedup vs baseline: 2.1460x; 2.1341x over previous
import jax
import jax.numpy as jnp
from jax import lax
from jax.experimental import pallas as pl

B, F, N, K = 8, 64, 256, 512


def _split(a):
    hi = a.astype(jnp.bfloat16)
    lo = (a - hi.astype(jnp.float32)).astype(jnp.bfloat16)
    return hi, lo


def _bdot(a, b, dims):
    return lax.dot_general(a, b, (dims, ((), ())),
                           preferred_element_type=jnp.float32)


NB = 128


def _vq_body(x_ref, emb_ref, q_ref, dist_ref):
    xb = x_ref[0]
    emb = emb_ref[...]
    xh, xl = _split(xb)
    eh, el = _split(emb)
    cd = ((1,), (0,))
    dotT = (_bdot(eh, xh, cd) + _bdot(eh, xl, cd)
            + _bdot(el, xh, cd))
    c2 = jnp.sum(emb * emb, axis=1, keepdims=True)
    x2 = jnp.sum(xb * xb, axis=0, keepdims=True)
    gT = c2 - 2.0 * dotT
    dist_ref[0] = (gT + x2).T
    minv = jnp.min(gT, axis=0, keepdims=True)
    iota = lax.broadcasted_iota(jnp.int32, (K, NB), 0)
    idx = jnp.min(jnp.where(gT == minv, iota, K), axis=0, keepdims=True)
    ohT = (iota == idx).astype(jnp.bfloat16)
    cq = ((0,), (0,))
    q_ref[0] = _bdot(eh, ohT, cq) + _bdot(el, ohT, cq)


def kernel(x, emb_weight):
    q, dist = pl.pallas_call(
        _vq_body,
        grid=(B, N // NB),
        in_specs=[
            pl.BlockSpec((1, F, NB), lambda b, t: (b, 0, t)),
            pl.BlockSpec((K, F), lambda b, t: (0, 0)),
        ],
        out_specs=[
            pl.BlockSpec((1, F, NB), lambda b, t: (b, 0, t)),
            pl.BlockSpec((1, NB, K), lambda b, t: (b, t, 0)),
        ],
        out_shape=[
            jax.ShapeDtypeStruct((B, F, N), jnp.float32),
            jax.ShapeDtypeStruct((B, N, K), jnp.float32),
        ],
    )(x, emb_weight)
    return q, dist


# speedup vs baseline: 3.0043x; 1.3999x over previous
import jax
import jax.numpy as jnp
from jax import lax
from jax.experimental import pallas as pl

B, F, N, K = 8, 64, 256, 512


def _split(a):
    hi = a.astype(jnp.bfloat16)
    lo = (a - hi.astype(jnp.float32)).astype(jnp.bfloat16)
    return hi, lo


def _bdot(a, b, dims):
    return lax.dot_general(a, b, (dims, ((), ())),
                           preferred_element_type=jnp.float32)


G = 4


def _vq_body(x_ref, emb_ref, q_ref, dist_ref):
  emb = emb_ref[...]
  eh, el = _split(emb)
  c2 = jnp.sum(emb * emb, axis=1, keepdims=True)
  for bi in range(G):
    xb = x_ref[bi]
    xh, xl = _split(xb)
    cd = ((1,), (0,))
    dotT = (_bdot(eh, xh, cd) + _bdot(eh, xl, cd)
            + _bdot(el, xh, cd))
    x2 = jnp.sum(xb * xb, axis=0, keepdims=True)
    gT = c2 - 2.0 * dotT
    dist_ref[bi] = (gT + x2).T
    minv = jnp.min(gT, axis=0, keepdims=True)
    iota = lax.broadcasted_iota(jnp.int32, (K, N), 0)
    idx = jnp.min(jnp.where(gT == minv, iota, K), axis=0, keepdims=True)
    ohT = (iota == idx).astype(jnp.bfloat16)
    cq = ((0,), (0,))
    q_ref[bi] = _bdot(eh, ohT, cq) + _bdot(el, ohT, cq)


def kernel(x, emb_weight):
    q, dist = pl.pallas_call(
        _vq_body,
        grid=(B // G,),
        in_specs=[
            pl.BlockSpec((G, F, N), lambda b: (b, 0, 0)),
            pl.BlockSpec((K, F), lambda b: (0, 0)),
        ],
        out_specs=[
            pl.BlockSpec((G, F, N), lambda b: (b, 0, 0)),
            pl.BlockSpec((G, N, K), lambda b: (b, 0, 0)),
        ],
        out_shape=[
            jax.ShapeDtypeStruct((B, F, N), jnp.float32),
            jax.ShapeDtypeStruct((B, N, K), jnp.float32),
        ],
    )(x, emb_weight)
    return q, dist
